# baseline (speedup 1.0000x reference)
import functools

import jax
import jax.numpy as jnp
from jax import lax
from jax.experimental import pallas as pl
from jax.experimental.pallas import tpu as pltpu
from jax.experimental.pallas import tpu_sc as plsc

BATCH = 16384
D = 64
IDX_CHUNK = 128


@functools.lru_cache(maxsize=1)
def _sc_gather_fn():
    info = plsc.get_sparse_core_info()
    nw = info.num_cores * info.num_subcores
    b_per_w = BATCH // nw
    n_chunks = b_per_w // IDX_CHUNK
    mesh = plsc.VectorSubcoreMesh(core_axis_name="c", subcore_axis_name="s")

    def body(seq_idx_hbm, mut_idx_hbm, seq_tab, mut_tab,
             seq_out, mut_out, idx_sv, idx_mv, rows_s, rows_m, sem):
        wid = lax.axis_index("s") * info.num_cores + lax.axis_index("c")
        base = wid * b_per_w
        pltpu.sync_copy(seq_idx_hbm.at[wid], idx_sv)
        pltpu.sync_copy(mut_idx_hbm.at[wid], idx_mv)
        copies = []
        for j in range(n_chunks):
            copies.append(pltpu.async_copy(
                seq_tab.at[idx_sv.at[j]],
                rows_s.at[pl.ds(j * IDX_CHUNK, IDX_CHUNK)], sem))
            copies.append(pltpu.async_copy(
                mut_tab.at[idx_mv.at[j]],
                rows_m.at[pl.ds(j * IDX_CHUNK, IDX_CHUNK)], sem))
        for c in copies:
            c.wait()
        pltpu.sync_copy(rows_s, seq_out.at[pl.ds(base, b_per_w)])
        pltpu.sync_copy(rows_m, mut_out.at[pl.ds(base, b_per_w)])

    return pl.kernel(
        body,
        out_type=[jax.ShapeDtypeStruct((BATCH, D), jnp.float32),
                  jax.ShapeDtypeStruct((BATCH, D), jnp.float32)],
        mesh=mesh,
        compiler_params=pltpu.CompilerParams(use_tc_tiling_on_sc=False),
        scratch_types=[
            pltpu.VMEM((n_chunks, IDX_CHUNK), jnp.int32),
            pltpu.VMEM((n_chunks, IDX_CHUNK), jnp.int32),
            pltpu.VMEM((b_per_w, D), jnp.float32),
            pltpu.VMEM((b_per_w, D), jnp.float32),
            pltpu.SemaphoreType.DMA,
        ],
    ), nw, n_chunks


def _mlp_body(seq_ref, mut_ref, w1a_ref, w1b_ref, b1_ref, w2_ref, b2_ref,
              o_ref):
    h = jnp.dot(seq_ref[...], w1a_ref[...], preferred_element_type=jnp.float32)
    h = h + jnp.dot(mut_ref[...], w1b_ref[...],
                    preferred_element_type=jnp.float32)
    h = jnp.maximum(h + b1_ref[...], 0.0)
    z = jnp.sum(h * w2_ref[...], axis=1, keepdims=True) + b2_ref[...]
    o_ref[...] = jax.nn.sigmoid(z)


def kernel(x, seq_emb, mut_emb, W1, b1, W2, b2):
    gather, nw, n_chunks = _sc_gather_fn()
    b_per_w = BATCH // nw
    xi = x.astype(jnp.int32)
    seq_idx = xi[:, 0].reshape(nw, n_chunks, IDX_CHUNK)
    mut_idx = xi[:, 1].reshape(nw, n_chunks, IDX_CHUNK)
    seq_rows, mut_rows = gather(seq_idx, mut_idx, seq_emb, mut_emb)

    blk = 2048
    grid = (BATCH // blk,)
    out = pl.pallas_call(
        _mlp_body,
        grid=grid,
        in_specs=[
            pl.BlockSpec((blk, D), lambda i: (i, 0)),
            pl.BlockSpec((blk, D), lambda i: (i, 0)),
            pl.BlockSpec((D, 2 * D), lambda i: (0, 0)),
            pl.BlockSpec((D, 2 * D), lambda i: (0, 0)),
            pl.BlockSpec((1, 2 * D), lambda i: (0, 0)),
            pl.BlockSpec((1, 2 * D), lambda i: (0, 0)),
            pl.BlockSpec((1, 1), lambda i: (0, 0)),
        ],
        out_specs=pl.BlockSpec((blk, 1), lambda i: (i, 0)),
        out_shape=jax.ShapeDtypeStruct((BATCH, 1), jnp.float32),
    )(seq_rows, mut_rows, W1[:D], W1[D:], b1.reshape(1, 2 * D),
      W2.reshape(1, 2 * D), b2.reshape(1, 1))
    return out
